# baseline (speedup 1.0000x reference)
import functools

import jax
import jax.numpy as jnp
from jax import lax
from jax.experimental import pallas as pl
from jax.experimental.pallas import tpu as pltpu, tpu_sc as plsc

_N_TOKENS = 32768
_D_MODEL = 768
_N_EXPERTS = 64
_LANES = 16

_F0 = 24576
_F1 = _N_TOKENS - _F0
_MM_BLK = 1024

_NW = 32
_CHUNK = 256
_GROUPS = _CHUNK // _LANES

_OUT_BLK = 4096


def _matmul_body(x_ref, w_ref, b_ref, o_ref):
    acc = lax.dot_general(
        w_ref[...], x_ref[...],
        dimension_numbers=(((1,), (1,)), ((), ())),
        preferred_element_type=jnp.float32,
    )
    o_ref[...] = acc + b_ref[:, 0:1]


def _logits_t(x, W, b2d, tok0, n_tok):
    blk0 = tok0 // _MM_BLK
    return pl.pallas_call(
        _matmul_body,
        grid=(n_tok // _MM_BLK,),
        in_specs=[
            pl.BlockSpec((_MM_BLK, _D_MODEL), lambda i: (i + blk0, 0)),
            pl.BlockSpec((_N_EXPERTS, _D_MODEL), lambda i: (0, 0)),
            pl.BlockSpec((_N_EXPERTS, 128), lambda i: (0, 0)),
        ],
        out_specs=pl.BlockSpec((_N_EXPERTS, _MM_BLK), lambda i: (0, i)),
        out_shape=jax.ShapeDtypeStruct((_N_EXPERTS, n_tok), jnp.float32),
    )(x, W, b2d)


def _make_route(n_tok):
    rows_per_w = n_tok // _NW
    nchunk = rows_per_w // _CHUNK

    def _route_body(lt_hbm, p1_hbm, i0_hbm, i1_hbm,
                    lt_v0, lt_v1, p1_v, i0_v, i1_v, lt_s0, lt_s1):
        wid = lax.axis_index("s") * 2 + lax.axis_index("c")
        tok0 = wid * rows_per_w
        lane = lax.iota(jnp.int32, _LANES)
        lt_vs = (lt_v0, lt_v1)
        lt_sems = (lt_s0, lt_s1)

        lt_dma = [None] * nchunk
        lt_dma[0] = pltpu.async_copy(
            lt_hbm.at[:, pl.ds(tok0, _CHUNK)], lt_v0, lt_s0)

        for c in range(nchunk):
            p = c & 1
            lt_v = lt_vs[p]
            base = tok0 + c * _CHUNK
            lt_dma[c].wait()
            if c + 1 < nchunk:
                lt_dma[c + 1] = pltpu.async_copy(
                    lt_hbm.at[:, pl.ds(base + _CHUNK, _CHUNK)],
                    lt_vs[p ^ 1], lt_sems[p ^ 1])

            def group_body(g, _):
                m0 = jnp.full((_LANES,), -jnp.inf, jnp.float32)
                m1 = jnp.full((_LANES,), -jnp.inf, jnp.float32)
                i0 = jnp.zeros((_LANES,), jnp.int32)
                i1 = jnp.zeros((_LANES,), jnp.int32)
                off = g * _LANES
                for e in range(_N_EXPERTS):
                    v = lt_v[e, pl.ds(off, _LANES)]
                    ev = jnp.full((_LANES,), e, jnp.int32)
                    gt0 = v > m0
                    gt1 = v > m1
                    i1 = jnp.where(gt0, i0, jnp.where(gt1, ev, i1))
                    m1 = jnp.where(gt0, m0, jnp.where(gt1, v, m1))
                    i0 = jnp.where(gt0, ev, i0)
                    m0 = jnp.where(gt0, v, m0)

                d = jnp.exp(m1 - m0)
                p1 = d / (d + 1.0)

                coff = c * _CHUNK + off
                p1_v[pl.ds(coff, _LANES)] = p1
                i0_v[pl.ds(coff, _LANES)] = i0
                i1_v[pl.ds(coff, _LANES)] = i1
                return 0

            lax.fori_loop(0, _GROUPS, group_body, 0)

        pltpu.sync_copy(p1_v, p1_hbm.at[pl.ds(tok0, rows_per_w)])
        pltpu.sync_copy(i0_v, i0_hbm.at[pl.ds(tok0, rows_per_w)])
        pltpu.sync_copy(i1_v, i1_hbm.at[pl.ds(tok0, rows_per_w)])

    return pl.kernel(
        _route_body,
        mesh=plsc.VectorSubcoreMesh(core_axis_name="c", subcore_axis_name="s"),
        out_type=(
            jax.ShapeDtypeStruct((n_tok,), jnp.float32),
            jax.ShapeDtypeStruct((n_tok,), jnp.int32),
            jax.ShapeDtypeStruct((n_tok,), jnp.int32),
        ),
        scratch_types=[
            pltpu.VMEM((_N_EXPERTS, _CHUNK), jnp.float32),
            pltpu.VMEM((_N_EXPERTS, _CHUNK), jnp.float32),
            pltpu.VMEM((rows_per_w,), jnp.float32),
            pltpu.VMEM((rows_per_w,), jnp.int32),
            pltpu.VMEM((rows_per_w,), jnp.int32),
            pltpu.SemaphoreType.DMA,
            pltpu.SemaphoreType.DMA,
        ],
        compiler_params=pltpu.CompilerParams(needs_layout_passes=False),
    )


_route_full = _make_route(_N_TOKENS)


def _mat_body(p1_ref, i0_ref, i1_ref, gate_ref, idx_ref):
    p1 = p1_ref[...]
    i0 = i0_ref[...]
    i1 = i1_ref[...]
    e = lax.broadcasted_iota(jnp.int32, (_OUT_BLK, _N_EXPERTS), 1)
    i0b = i0[:, None]
    i1b = i1[:, None]
    p1b = p1[:, None]
    gate_ref[...] = jnp.where(
        e == i0b, 1.0 - p1b, jnp.where(e == i1b, p1b, 0.0))
    idx_ref[...] = jnp.concatenate([i0b, i1b], axis=1)


def _mat_body_alias(p1_ref, i0_ref, i1_ref, _g_in, _x_in, gate_ref, idx_ref):
    _mat_body(p1_ref, i0_ref, i1_ref, gate_ref, idx_ref)


_OUT_SHAPES = [
    jax.ShapeDtypeStruct((_N_TOKENS, _N_EXPERTS), jnp.float32),
    jax.ShapeDtypeStruct((_N_TOKENS, 2), jnp.int32),
]
_IN_BLOCKS = [
    pl.BlockSpec((_OUT_BLK,), lambda i: (i,)),
    pl.BlockSpec((_OUT_BLK,), lambda i: (i,)),
    pl.BlockSpec((_OUT_BLK,), lambda i: (i,)),
]


def _materialize0(p1, i0, i1):
    return pl.pallas_call(
        _mat_body,
        grid=(_F0 // _OUT_BLK,),
        in_specs=_IN_BLOCKS,
        out_specs=[
            pl.BlockSpec((_OUT_BLK, _N_EXPERTS), lambda i: (i, 0)),
            pl.BlockSpec((_OUT_BLK, 2), lambda i: (i, 0)),
        ],
        out_shape=_OUT_SHAPES,
    )(p1, i0, i1)


def _materialize1(p1, i0, i1, gate_prev, idx_prev):
    nblk0 = _F0 // _OUT_BLK
    in_blocks = [
        pl.BlockSpec((_OUT_BLK,), lambda i: (i + nblk0,)),
        pl.BlockSpec((_OUT_BLK,), lambda i: (i + nblk0,)),
        pl.BlockSpec((_OUT_BLK,), lambda i: (i + nblk0,)),
    ]
    return pl.pallas_call(
        _mat_body_alias,
        grid=(_F1 // _OUT_BLK,),
        in_specs=in_blocks + [
            pl.BlockSpec(memory_space=pl.ANY),
            pl.BlockSpec(memory_space=pl.ANY),
        ],
        out_specs=[
            pl.BlockSpec((_OUT_BLK, _N_EXPERTS), lambda i: (i + nblk0, 0)),
            pl.BlockSpec((_OUT_BLK, 2), lambda i: (i + nblk0, 0)),
        ],
        out_shape=_OUT_SHAPES,
        input_output_aliases={3: 0, 4: 1},
    )(p1, i0, i1, gate_prev, idx_prev)


@jax.jit
def kernel(x, W, b):
    b2d = jnp.broadcast_to(b[:, None], (_N_EXPERTS, 128))
    lt = _logits_t(x, W, b2d, 0, _N_TOKENS)
    p1, i0, i1 = _route_full(lt)
    gate0, idx0 = _materialize0(p1, i0, i1)
    gate, idx = _materialize1(p1, i0, i1, gate0, idx0)
    return (gate, idx)


# speedup vs baseline: 1.1243x; 1.1243x over previous
import functools

import jax
import jax.numpy as jnp
from jax import lax
from jax.experimental import pallas as pl
from jax.experimental.pallas import tpu as pltpu, tpu_sc as plsc

_N_TOKENS = 32768
_D_MODEL = 768
_N_EXPERTS = 64
_LANES = 16

_F0 = 24576
_F1 = _N_TOKENS - _F0
_MM_BLK = 1024

_NW = 32
_CHUNK = 256
_GROUPS = _CHUNK // _LANES

_OUT_BLK = 4096

_PSCALE = float(1 << 21)
_PMAX = (1 << 20) - 1


def _matmul_body(x_ref, w_ref, b_ref, o_ref):
    acc = lax.dot_general(
        w_ref[...], x_ref[...],
        dimension_numbers=(((1,), (1,)), ((), ())),
        preferred_element_type=jnp.float32,
    )
    o_ref[...] = acc + b_ref[:, 0:1]


def _logits_t(x, W, b2d, tok0, n_tok):
    blk0 = tok0 // _MM_BLK
    return pl.pallas_call(
        _matmul_body,
        grid=(n_tok // _MM_BLK,),
        in_specs=[
            pl.BlockSpec((_MM_BLK, _D_MODEL), lambda i: (i + blk0, 0)),
            pl.BlockSpec((_N_EXPERTS, _D_MODEL), lambda i: (0, 0)),
            pl.BlockSpec((_N_EXPERTS, 128), lambda i: (0, 0)),
        ],
        out_specs=pl.BlockSpec((_N_EXPERTS, _MM_BLK), lambda i: (0, i)),
        out_shape=jax.ShapeDtypeStruct((_N_EXPERTS, n_tok), jnp.float32),
    )(x, W, b2d)


def _make_route(n_tok):
    rows_per_w = n_tok // _NW
    nchunk = rows_per_w // _CHUNK

    def _route_body(lt_hbm, w_hbm, lt_v0, lt_v1, w_v, lt_s0, lt_s1):
        wid = lax.axis_index("s") * 2 + lax.axis_index("c")
        tok0 = wid * rows_per_w
        lt_vs = (lt_v0, lt_v1)
        lt_sems = (lt_s0, lt_s1)

        lt_dma = [None] * nchunk
        lt_dma[0] = pltpu.async_copy(
            lt_hbm.at[:, pl.ds(tok0, _CHUNK)], lt_v0, lt_s0)

        for c in range(nchunk):
            p = c & 1
            lt_v = lt_vs[p]
            base = tok0 + c * _CHUNK
            lt_dma[c].wait()
            if c + 1 < nchunk:
                lt_dma[c + 1] = pltpu.async_copy(
                    lt_hbm.at[:, pl.ds(base + _CHUNK, _CHUNK)],
                    lt_vs[p ^ 1], lt_sems[p ^ 1])

            def group_body(g, _):
                m0 = jnp.full((_LANES,), -jnp.inf, jnp.float32)
                m1 = jnp.full((_LANES,), -jnp.inf, jnp.float32)
                i0 = jnp.zeros((_LANES,), jnp.int32)
                i1 = jnp.zeros((_LANES,), jnp.int32)
                off = g * _LANES
                for e in range(_N_EXPERTS):
                    v = lt_v[e, pl.ds(off, _LANES)]
                    ev = jnp.full((_LANES,), e, jnp.int32)
                    gt0 = v > m0
                    gt1 = v > m1
                    i1 = jnp.where(gt0, i0, jnp.where(gt1, ev, i1))
                    m1 = jnp.where(gt0, m0, jnp.where(gt1, v, m1))
                    i0 = jnp.where(gt0, ev, i0)
                    m0 = jnp.where(gt0, v, m0)

                d = jnp.exp(m1 - m0)
                p1 = d / (d + 1.0)
                p1q = jnp.minimum((p1 * _PSCALE).astype(jnp.int32), _PMAX)
                w = i0 | (i1 << 6) | (p1q << 12)

                w_v[pl.ds(c * _CHUNK + off, _LANES)] = w
                return 0

            lax.fori_loop(0, _GROUPS, group_body, 0)

        pltpu.sync_copy(w_v, w_hbm.at[pl.ds(tok0, rows_per_w)])

    return pl.kernel(
        _route_body,
        mesh=plsc.VectorSubcoreMesh(core_axis_name="c", subcore_axis_name="s"),
        out_type=jax.ShapeDtypeStruct((n_tok,), jnp.int32),
        scratch_types=[
            pltpu.VMEM((_N_EXPERTS, _CHUNK), jnp.float32),
            pltpu.VMEM((_N_EXPERTS, _CHUNK), jnp.float32),
            pltpu.VMEM((rows_per_w,), jnp.int32),
            pltpu.SemaphoreType.DMA,
            pltpu.SemaphoreType.DMA,
        ],
        compiler_params=pltpu.CompilerParams(needs_layout_passes=False),
    )


_route0 = _make_route(_F0)
_route1 = _make_route(_F1)


def _mat_body(w_ref, gate_ref, idx_ref):
    w = w_ref[...]
    wb = w[:, None]
    wbb = jnp.broadcast_to(wb, (_OUT_BLK, _N_EXPERTS))
    e = lax.broadcasted_iota(jnp.int32, (_OUT_BLK, _N_EXPERTS), 1)
    i0b = wbb & 63
    i1b = lax.shift_right_logical(wbb, 6) & 63
    p1b = lax.shift_right_logical(wbb, 12).astype(jnp.float32) * (1.0 / _PSCALE)
    gate_ref[...] = jnp.where(
        e == i0b, 1.0 - p1b, jnp.where(e == i1b, p1b, 0.0))
    i0c = wb & 63
    i1c = lax.shift_right_logical(wb, 6) & 63
    idx_ref[...] = jnp.concatenate([i0c, i1c], axis=1)


def _mat_body_alias(w_ref, _g_in, _x_in, gate_ref, idx_ref):
    _mat_body(w_ref, gate_ref, idx_ref)


_OUT_SHAPES = [
    jax.ShapeDtypeStruct((_N_TOKENS, _N_EXPERTS), jnp.float32),
    jax.ShapeDtypeStruct((_N_TOKENS, 2), jnp.int32),
]


def _materialize0(w):
    return pl.pallas_call(
        _mat_body,
        grid=(_F0 // _OUT_BLK,),
        in_specs=[pl.BlockSpec((_OUT_BLK,), lambda i: (i,))],
        out_specs=[
            pl.BlockSpec((_OUT_BLK, _N_EXPERTS), lambda i: (i, 0)),
            pl.BlockSpec((_OUT_BLK, 2), lambda i: (i, 0)),
        ],
        out_shape=_OUT_SHAPES,
    )(w)


def _materialize1(w, gate_prev, idx_prev):
    nblk0 = _F0 // _OUT_BLK
    return pl.pallas_call(
        _mat_body_alias,
        grid=(_F1 // _OUT_BLK,),
        in_specs=[
            pl.BlockSpec((_OUT_BLK,), lambda i: (i,)),
            pl.BlockSpec(memory_space=pl.ANY),
            pl.BlockSpec(memory_space=pl.ANY),
        ],
        out_specs=[
            pl.BlockSpec((_OUT_BLK, _N_EXPERTS), lambda i: (i + nblk0, 0)),
            pl.BlockSpec((_OUT_BLK, 2), lambda i: (i + nblk0, 0)),
        ],
        out_shape=_OUT_SHAPES,
        input_output_aliases={1: 0, 2: 1},
    )(w, gate_prev, idx_prev)


@jax.jit
def kernel(x, W, b):
    b2d = jnp.broadcast_to(b[:, None], (_N_EXPERTS, 128))
    lt0 = _logits_t(x, W, b2d, 0, _F0)
    w0 = _route0(lt0)
    lt1 = _logits_t(x, W, b2d, _F0, _F1)
    w1 = _route1(lt1)
    gate0, idx0 = _materialize0(w0)
    gate, idx = _materialize1(w1, gate0, idx0)
    return (gate, idx)
